# baseline (speedup 1.0000x reference)
import functools

import jax
import jax.numpy as jnp
from jax import lax
from jax.experimental import pallas as pl
from jax.experimental.pallas import tpu as pltpu
from jax.experimental.pallas import tpu_sc as plsc

VOCAB = 1000000
D = 64
ROWS = 4096
TOK = 200
TOK_H = TOK // 2
NC, NS, LANES = 2, 16, 16
NW = NC * NS
R_PER_W = ROWS // NW
NBUF = 4
NGROUP = R_PER_W // NBUF
SCALE = 8.0

_mesh = plsc.VectorSubcoreMesh(
    core_axis_name="c", subcore_axis_name="s", num_cores=NC, num_subcores=NS
)

_scratch = (
    [pltpu.VMEM((R_PER_W, TOK_H), jnp.int32)]
    + [pltpu.VMEM((1, TOK_H, D), jnp.float32) for _ in range(2 * NBUF)]
    + [pltpu.SemaphoreType.DMA for _ in range(2 * NBUF)]
)


@functools.partial(
    pl.kernel,
    out_type=jax.ShapeDtypeStruct((ROWS, TOK_H, D), jnp.float32),
    mesh=_mesh,
    scratch_types=_scratch,
    compiler_params=pltpu.CompilerParams(use_tc_tiling_on_sc=False),
)
def _emb_kernel(x_hbm, table_hbm, out_hbm, idx, *bufs):
    gbuf = bufs[:NBUF]
    obuf = bufs[NBUF:2 * NBUF]
    gsem = bufs[2 * NBUF:3 * NBUF]
    wsem = bufs[3 * NBUF:]

    wid = lax.axis_index("s") * NC + lax.axis_index("c")
    row0 = wid * R_PER_W

    pltpu.sync_copy(x_hbm.at[pl.ds(row0, R_PER_W)], idx)

    def start_gather(b, r):
        pltpu.async_copy(table_hbm.at[idx.at[r]], gbuf[b].at[0], gsem[b])

    def wait_gather(b, r):
        pltpu.make_async_copy(
            table_hbm.at[idx.at[r]], gbuf[b].at[0], gsem[b]
        ).wait()

    def start_write(b, r):
        pltpu.async_copy(obuf[b], out_hbm.at[pl.ds(row0 + r, 1)], wsem[b])

    def wait_write(b, r):
        pltpu.make_async_copy(
            obuf[b], out_hbm.at[pl.ds(row0 + r, 1)], wsem[b]
        ).wait()

    def scale(b):
        src, dst = gbuf[b], obuf[b]

        @plsc.parallel_loop(0, TOK_H, unroll=8)
        def _(t):
            for j in range(D // LANES):
                sl = pl.ds(j * LANES, LANES)
                dst[0, t, sl] = src[0, t, sl] * SCALE

    for b in range(NBUF):
        start_gather(b, b)

    @pl.loop(0, NGROUP)
    def _(t):
        for b in range(NBUF):
            r = t * NBUF + b
            wait_gather(b, r)

            @pl.when(t > 0)
            def _():
                wait_write(b, r)

            scale(b)
            start_write(b, r)

            @pl.when(t < NGROUP - 1)
            def _():
                start_gather(b, r + NBUF)

    for b in range(NBUF):
        wait_write(b, 0)


def kernel(x, table):
    xi = x.astype(jnp.int32)
    o1 = _emb_kernel(xi[:, :TOK_H], table)
    o2 = _emb_kernel(xi[:, TOK_H:], table)
    return jnp.concatenate([o1, o2], axis=1)


# speedup vs baseline: 1.2054x; 1.2054x over previous
import functools

import jax
import jax.numpy as jnp
from jax import lax
from jax.experimental import pallas as pl
from jax.experimental.pallas import tpu as pltpu
from jax.experimental.pallas import tpu_sc as plsc

VOCAB = 1000000
D = 64
ROWS = 4096
TOK = 200
NC, NS, LANES = 2, 16, 16
NW = NC * NS
R_PER_W = ROWS // NW
NBUF = 4
NGROUP = R_PER_W // NBUF
SCALE = 8.0

_mesh = plsc.VectorSubcoreMesh(
    core_axis_name="c", subcore_axis_name="s", num_cores=NC, num_subcores=NS
)

_scratch = (
    [pltpu.VMEM((R_PER_W, 128), jnp.int32),
     pltpu.VMEM((R_PER_W, TOK - 128), jnp.int32)]
    + [pltpu.VMEM((1, TOK, D), jnp.float32) for _ in range(2 * NBUF)]
    + [pltpu.SemaphoreType.DMA for _ in range(2 * NBUF)]
)


@functools.partial(
    pl.kernel,
    out_type=jax.ShapeDtypeStruct((ROWS, TOK, D), jnp.float32),
    mesh=_mesh,
    scratch_types=_scratch,
    compiler_params=pltpu.CompilerParams(use_tc_tiling_on_sc=False),
)
def _emb_kernel(x_hbm, table_hbm, out_hbm, idxa, idxb, *bufs):
    gbuf = bufs[:NBUF]
    obuf = bufs[NBUF:2 * NBUF]
    gsem = bufs[2 * NBUF:3 * NBUF]
    wsem = bufs[3 * NBUF:]

    wid = lax.axis_index("s") * NC + lax.axis_index("c")
    row0 = wid * R_PER_W

    pltpu.sync_copy(x_hbm.at[pl.ds(row0, R_PER_W), pl.ds(0, 128)], idxa)
    pltpu.sync_copy(x_hbm.at[pl.ds(row0, R_PER_W), pl.ds(128, TOK - 128)],
                    idxb)

    splits = ((idxa, 0, 128), (idxb, 128, TOK - 128))

    def start_gather(b, r):
        for ref, off, n in splits:
            pltpu.async_copy(
                table_hbm.at[ref.at[r]],
                gbuf[b].at[0, pl.ds(off, n)],
                gsem[b],
            )

    def wait_gather(b, r):
        for ref, off, n in splits:
            pltpu.make_async_copy(
                table_hbm.at[ref.at[r]],
                gbuf[b].at[0, pl.ds(off, n)],
                gsem[b],
            ).wait()

    def start_write(b, r):
        pltpu.async_copy(obuf[b], out_hbm.at[pl.ds(row0 + r, 1)], wsem[b])

    def wait_write(b, r):
        pltpu.make_async_copy(
            obuf[b], out_hbm.at[pl.ds(row0 + r, 1)], wsem[b]
        ).wait()

    def scale(b):
        src, dst = gbuf[b], obuf[b]

        @plsc.parallel_loop(0, TOK, unroll=8)
        def _(t):
            for j in range(D // LANES):
                sl = pl.ds(j * LANES, LANES)
                dst[0, t, sl] = src[0, t, sl] * SCALE

    for b in range(NBUF):
        start_gather(b, b)

    @pl.loop(0, NGROUP)
    def _(t):
        for b in range(NBUF):
            r = t * NBUF + b
            wait_gather(b, r)

            @pl.when(t > 0)
            def _():
                wait_write(b, r)

            scale(b)
            start_write(b, r)

            @pl.when(t < NGROUP - 1)
            def _():
                start_gather(b, r + NBUF)

    for b in range(NBUF):
        wait_write(b, 0)


def kernel(x, table):
    return _emb_kernel(x.astype(jnp.int32), table)
